# baseline (speedup 1.0000x reference)
import functools

import jax
import jax.numpy as jnp
from jax.experimental import pallas as pl
from jax.experimental.pallas import tpu as pltpu

_BLOCK_ROWS = 1024
_K = 5


def _ce(a, b):
    return jnp.minimum(a, b), jnp.maximum(a, b)


def _sort_mountain5(c):
    c0, c1, c2, c3, c4 = c
    c0, c4 = _ce(c0, c4)
    c1, c3 = _ce(c1, c3)
    c1, c4 = _ce(c1, c4)
    c2, c4 = _ce(c2, c4)
    c3, c4 = _ce(c3, c4)
    return [c0, c1, c2, c3, c4]


def _merge22(p, q):
    w1, t1 = _ce(p[0], q[0])
    t2, w4 = _ce(p[1], q[1])
    w2, w3 = _ce(t1, t2)
    return [w1, w2, w3, w4]


def _merge44_keep5(a, b):
    c = [a[0], jnp.minimum(a[1], b[3]), jnp.minimum(a[2], b[2]),
         jnp.minimum(a[3], b[1]), b[0]]
    return _sort_mountain5(c)


def _merge55_keep5(a, b, sort=True):
    c = [jnp.minimum(a[0], b[4]), jnp.minimum(a[1], b[3]),
         jnp.minimum(a[2], b[2]), jnp.minimum(a[3], b[1]),
         jnp.minimum(a[4], b[0])]
    return _sort_mountain5(c) if sort else c


def _knn_moments_kernel(rows_ref, full_ref, out_ref, sqy_ref, *, t, k,
                        n_blocks):
    i = pl.program_id(0)

    x = rows_ref[...]
    y = full_ref[...]
    sqx = jnp.sum(x * x, axis=1, keepdims=True)

    @pl.when(i == 0)
    def _fill_sqy():
        sqy_ref[...] = jnp.sum(y * y, axis=1, keepdims=True).T

    sqy = sqy_ref[...]
    xy = jax.lax.dot_general(
        (-2.0 * x).astype(jnp.bfloat16), y.astype(jnp.bfloat16),
        dimension_numbers=(((1,), (1,)), ((), ())),
        preferred_element_type=jnp.float32,
    )

    chunk = 128
    n_chunks = t // chunk

    def d2_chunk(c):
        sl = slice(c * chunk, (c + 1) * chunk)
        return sqy[:, sl] + xy[:, sl]

    vs = [d2_chunk(c) for c in range(n_chunks)]
    pairs = [list(_ce(vs[2 * c], vs[2 * c + 1])) for c in range(n_chunks // 2)]
    s4 = [_merge22(pairs[2 * c], pairs[2 * c + 1]) for c in range(len(pairs) // 2)]
    s5 = [_merge44_keep5(s4[2 * c], s4[2 * c + 1]) for c in range(len(s4) // 2)]
    while len(s5) > 1:
        s5 = [_merge55_keep5(s5[2 * c], s5[2 * c + 1]) for c in range(len(s5) // 2)]
    st = s5[0]

    stT = [a.T for a in st]
    h = 128
    while h > 8:
        h //= 2
        stT = _merge55_keep5([a[:h] for a in stT], [a[h:] for a in stT])
    for sh in (4, 2, 1):
        partner = [pltpu.roll(a, sh, 0) for a in stT]
        stT = _merge55_keep5(stT, partner, sort=(sh > 1))

    sqxT = sqx.T
    macc = jnp.zeros_like(stT[0])
    m2acc = jnp.zeros_like(stT[0])
    for j in range(k):
        m = jnp.sqrt(jnp.maximum(sqxT + stT[j], 0.0))
        macc = macc + m
        m2acc = m2acc + m * m
    row = jax.lax.broadcasted_iota(jnp.int32, stT[0].shape, 0)
    s = jnp.sum(jnp.where(row == 0, macc, 0.0))
    ss = jnp.sum(jnp.where(row == 0, m2acc, 0.0))

    lvec = jax.lax.broadcasted_iota(jnp.int32, (1, 128), 1)
    vec = jnp.where(lvec == 0, s, jnp.where(lvec == 1, ss, 0.0))

    @pl.when(i == 0)
    def _init():
        out_ref[...] = vec

    @pl.when(i > 0)
    def _acc():
        out_ref[...] += vec


@jax.jit
def kernel(latent):
    if latent.shape[0] < 2:
        return jnp.asarray(0.0, dtype=latent.dtype)
    b, t, d = latent.shape
    sample = latent[0].astype(jnp.float32)
    k = min(_K, t - 1)
    n_blocks = t // _BLOCK_ROWS

    moments = pl.pallas_call(
        functools.partial(_knn_moments_kernel, t=t, k=k, n_blocks=n_blocks),
        grid=(n_blocks,),
        in_specs=[
            pl.BlockSpec((_BLOCK_ROWS, d), lambda i: (i, 0)),
            pl.BlockSpec((t, d), lambda i: (0, 0)),
        ],
        out_specs=pl.BlockSpec((1, 128), lambda i: (0, 0)),
        out_shape=jax.ShapeDtypeStruct((1, 128), jnp.float32),
        scratch_shapes=[pltpu.VMEM((1, t), jnp.float32)],
    )(sample, sample)

    n = jnp.float32(t * k)
    s = moments[0, 0]
    ss = moments[0, 1]
    var = (ss - s * s / n) / (n - 1.0)
    return (-jnp.sqrt(jnp.maximum(var, 0.0))).astype(latent.dtype)


# speedup vs baseline: 1.0384x; 1.0384x over previous
import functools

import jax
import jax.numpy as jnp
from jax.experimental import pallas as pl
from jax.experimental.pallas import tpu as pltpu

_BLOCK_ROWS = 2048
_K = 5


def _ce(a, b):
    return jnp.minimum(a, b), jnp.maximum(a, b)


def _sort_mountain5(c):
    c0, c1, c2, c3, c4 = c
    c0, c4 = _ce(c0, c4)
    c1, c3 = _ce(c1, c3)
    c1, c4 = _ce(c1, c4)
    c2, c4 = _ce(c2, c4)
    c3, c4 = _ce(c3, c4)
    return [c0, c1, c2, c3, c4]


def _merge22(p, q):
    w1, t1 = _ce(p[0], q[0])
    t2, w4 = _ce(p[1], q[1])
    w2, w3 = _ce(t1, t2)
    return [w1, w2, w3, w4]


def _merge44_keep5(a, b):
    c = [a[0], jnp.minimum(a[1], b[3]), jnp.minimum(a[2], b[2]),
         jnp.minimum(a[3], b[1]), b[0]]
    return _sort_mountain5(c)


def _merge55_keep5(a, b, sort=True):
    c = [jnp.minimum(a[0], b[4]), jnp.minimum(a[1], b[3]),
         jnp.minimum(a[2], b[2]), jnp.minimum(a[3], b[1]),
         jnp.minimum(a[4], b[0])]
    return _sort_mountain5(c) if sort else c


def _knn_moments_kernel(rows_ref, full_ref, out_ref, sqy_ref, *, t, k,
                        n_blocks):
    i = pl.program_id(0)

    x = rows_ref[...]
    y = full_ref[...]
    sqx = jnp.sum(x * x, axis=1, keepdims=True)

    @pl.when(i == 0)
    def _fill_sqy():
        sqy_ref[...] = jnp.sum(y * y, axis=1, keepdims=True).T

    sqy = sqy_ref[...]
    xy = jax.lax.dot_general(
        (-2.0 * x).astype(jnp.bfloat16), y.astype(jnp.bfloat16),
        dimension_numbers=(((1,), (1,)), ((), ())),
        preferred_element_type=jnp.float32,
    )

    chunk = 128
    n_chunks = t // chunk

    def d2_chunk(c):
        sl = slice(c * chunk, (c + 1) * chunk)
        return sqy[:, sl] + xy[:, sl]

    vs = [d2_chunk(c) for c in range(n_chunks)]
    pairs = [list(_ce(vs[2 * c], vs[2 * c + 1])) for c in range(n_chunks // 2)]
    s4 = [_merge22(pairs[2 * c], pairs[2 * c + 1]) for c in range(len(pairs) // 2)]
    s5 = [_merge44_keep5(s4[2 * c], s4[2 * c + 1]) for c in range(len(s4) // 2)]
    while len(s5) > 1:
        s5 = [_merge55_keep5(s5[2 * c], s5[2 * c + 1]) for c in range(len(s5) // 2)]
    st = s5[0]

    stT = [a.T for a in st]
    h = 128
    while h > 8:
        h //= 2
        stT = _merge55_keep5([a[:h] for a in stT], [a[h:] for a in stT])
    for sh in (4, 2, 1):
        partner = [pltpu.roll(a, sh, 0) for a in stT]
        stT = _merge55_keep5(stT, partner, sort=(sh > 1))

    sqxT = sqx.T
    macc = jnp.zeros_like(stT[0])
    m2acc = jnp.zeros_like(stT[0])
    for j in range(k):
        m = jnp.sqrt(jnp.maximum(sqxT + stT[j], 0.0))
        macc = macc + m
        m2acc = m2acc + m * m
    row = jax.lax.broadcasted_iota(jnp.int32, stT[0].shape, 0)
    s = jnp.sum(jnp.where(row == 0, macc, 0.0))
    ss = jnp.sum(jnp.where(row == 0, m2acc, 0.0))

    lvec = jax.lax.broadcasted_iota(jnp.int32, (1, 128), 1)
    vec = jnp.where(lvec == 0, s, jnp.where(lvec == 1, ss, 0.0))

    @pl.when(i == 0)
    def _init():
        out_ref[...] = vec

    @pl.when(i > 0)
    def _acc():
        out_ref[...] += vec


@jax.jit
def kernel(latent):
    if latent.shape[0] < 2:
        return jnp.asarray(0.0, dtype=latent.dtype)
    b, t, d = latent.shape
    sample = latent[0].astype(jnp.float32)
    k = min(_K, t - 1)
    n_blocks = t // _BLOCK_ROWS

    moments = pl.pallas_call(
        functools.partial(_knn_moments_kernel, t=t, k=k, n_blocks=n_blocks),
        grid=(n_blocks,),
        in_specs=[
            pl.BlockSpec((_BLOCK_ROWS, d), lambda i: (i, 0)),
            pl.BlockSpec((t, d), lambda i: (0, 0)),
        ],
        out_specs=pl.BlockSpec((1, 128), lambda i: (0, 0)),
        out_shape=jax.ShapeDtypeStruct((1, 128), jnp.float32),
        scratch_shapes=[pltpu.VMEM((1, t), jnp.float32)],
    )(sample, sample)

    n = jnp.float32(t * k)
    s = moments[0, 0]
    ss = moments[0, 1]
    var = (ss - s * s / n) / (n - 1.0)
    return (-jnp.sqrt(jnp.maximum(var, 0.0))).astype(latent.dtype)
